# baseline (speedup 1.0000x reference)
import functools

import jax
import jax.numpy as jnp
from jax import lax
from jax.experimental import pallas as pl
from jax.experimental.pallas import tpu as pltpu
from jax.experimental.pallas import tpu_sc as plsc

E = 800000
N_NEI = 4
IN_C = 16
MID = 32
POOL = 32
CAT = 40

NW = 32
ROWS_W = E // NW
R = 256
NCHUNK = ROWS_W // R
R_TAIL = ROWS_W - NCHUNK * R

BM = 8000


def _proj_body(x_ref, w_ref, b_ref, s_ref, y_ref):
    x = x_ref[...]
    c = x.shape[1]
    s_ref[...] = (
        jnp.dot(x, w_ref[0:c, :], preferred_element_type=jnp.float32)
        + b_ref[...]
    )
    for j in range(N_NEI):
        y_ref[j] = jnp.dot(
            x, w_ref[(j + 1) * c:(j + 2) * c, :],
            preferred_element_type=jnp.float32,
        )


def _tc_projections(x, wstack, b):
    c = x.shape[1]
    s, ybig = pl.pallas_call(
        _proj_body,
        grid=(E // BM,),
        in_specs=[
            pl.BlockSpec((BM, c), lambda i: (i, 0)),
            pl.BlockSpec((5 * c, MID), lambda i: (0, 0)),
            pl.BlockSpec((1, MID), lambda i: (0, 0)),
        ],
        out_specs=(
            pl.BlockSpec((BM, MID), lambda i: (i, 0)),
            pl.BlockSpec((N_NEI, BM, MID), lambda i: (0, i, 0)),
        ),
        out_shape=(
            jax.ShapeDtypeStruct((E, MID), jnp.float32),
            jax.ShapeDtypeStruct((N_NEI, E, MID), jnp.float32),
        ),
    )(x, wstack, b.reshape(1, MID))
    return s, ybig.reshape(N_NEI * E, MID)


def _make_sc_combine(has_skip):
    mesh = plsc.VectorSubcoreMesh(core_axis_name="c", subcore_axis_name="s")

    def body(*refs):
        if has_skip:
            (s_hbm, ybig, he_hbm, skip_hbm, out_hbm,
             ilist, gb, sb, kb, sem) = refs
        else:
            (s_hbm, ybig, he_hbm, out_hbm,
             ilist, gb, sb, sem) = refs
            kb = None
        wid = lax.axis_index("s") * 2 + lax.axis_index("c")
        base = wid * ROWS_W

        iota = lax.iota(jnp.int32, 16)
        r4e = (iota & 3) * E

        def chunk(k, carry):
            off = base + jnp.minimum(k * R, ROWS_W - R)
            pltpu.sync_copy(he_hbm.at[pl.ds(off * 4, R * 4)], ilist)
            pltpu.sync_copy(s_hbm.at[pl.ds(off, R)], sb)
            if has_skip:
                pltpu.sync_copy(skip_hbm.at[pl.ds(off, R)], kb)

            def build(i, bcarry):
                sl = pl.ds(i * 16, 16)
                ilist[sl] = ilist[sl] + r4e
                return bcarry

            lax.fori_loop(0, R * 4 // 16, build, 0, unroll=8)

            pltpu.async_copy(ybig.at[ilist], gb, sem).wait()

            def row(r, rcarry):
                for c in (0, 16):
                    sl = pl.ds(c, 16)
                    v = (sb[r, sl] + gb[4 * r, sl] + gb[4 * r + 1, sl]
                         + gb[4 * r + 2, sl] + gb[4 * r + 3, sl])
                    v = jnp.maximum(v, 0.0)
                    if has_skip:
                        v = jnp.maximum(v + kb[r, sl], 0.0)
                    sb[r, sl] = v
                return rcarry

            lax.fori_loop(0, R, row, 0, unroll=2)
            pltpu.sync_copy(sb, out_hbm.at[pl.ds(off, R)])
            return carry

        nch = NCHUNK + (1 if R_TAIL else 0)
        lax.fori_loop(0, nch, chunk, 0)

    scratch = [
        pltpu.VMEM((N_NEI * R,), jnp.int32),
        pltpu.VMEM((N_NEI * R, MID), jnp.float32),
        pltpu.VMEM((R, MID), jnp.float32),
    ]
    if has_skip:
        scratch.append(pltpu.VMEM((R, MID), jnp.float32))
    scratch.append(pltpu.SemaphoreType.DMA)

    return functools.partial(
        pl.kernel,
        mesh=mesh,
        out_type=jax.ShapeDtypeStruct((E, MID), jnp.float32),
        scratch_types=scratch,
        compiler_params=pltpu.CompilerParams(use_tc_tiling_on_sc=False),
    )(body)


_sc_combine = _make_sc_combine(False)
_sc_combine_skip = _make_sc_combine(True)


def _pool_body(h_ref, wf_ref, bf_ref, o_ref):
    p = pl.program_id(0)
    m = jnp.mean(h_ref[...], axis=0).reshape(1, MID)
    part = jnp.dot(m, wf_ref[...], preferred_element_type=jnp.float32)

    @pl.when(p == 0)
    def _():
        o_ref[...] = bf_ref[...]

    o_ref[...] += part


def _pool_fc(h, wf, bf):
    seg = E // POOL
    out = pl.pallas_call(
        _pool_body,
        grid=(POOL,),
        in_specs=[
            pl.BlockSpec((seg, MID), lambda p: (p, 0)),
            pl.BlockSpec((MID, CAT), lambda p: (p, 0)),
            pl.BlockSpec((1, CAT), lambda p: (0, 0)),
        ],
        out_specs=pl.BlockSpec((1, CAT), lambda p: (0, 0)),
        out_shape=jax.ShapeDtypeStruct((1, CAT), jnp.float32),
    )(h, wf, bf.reshape(1, CAT))
    return out.reshape(CAT)


def _conv(x, he_flat, w, b, skip=None):
    s, ybig = _tc_projections(x, w, b)
    if skip is None:
        return _sc_combine(s, ybig, he_flat)
    return _sc_combine_skip(s, ybig, he_flat, skip)


def kernel(x, half_edges, W0, b0, W11, b11, W12, b12, W21, b21, W22, b22, Wf, bf):
    he_flat = half_edges.reshape(N_NEI * E)
    h = _conv(x, he_flat, W0, b0)
    for (wa, ba, wb, bb) in ((W11, b11, W12, b12), (W21, b21, W22, b22)):
        y = _conv(h, he_flat, wa, ba)
        h = _conv(y, he_flat, wb, bb, skip=h)
    return _pool_fc(h, Wf, bf)
